# baseline (speedup 1.0000x reference)
import functools

import jax
import jax.numpy as jnp
from jax import lax
from jax.experimental import pallas as pl
from jax.experimental.pallas import tpu as pltpu
from jax.experimental.pallas import tpu_sc as plsc

N_NODES = 10000
D_FEAT = 128
N_EDGES = 320000

NPAD = 10240
NROWS = NPAD // 128
RC = 79
ECP = RC * 128
NCHUNKS = N_EDGES // 128
NSLC = NPAD // 16

_mesh = plsc.VectorSubcoreMesh(core_axis_name="c", subcore_axis_name="s")


def _zero_shared(zbuf, shared, s):
    @plsc.parallel_loop(0, NSLC // 16, unroll=4)
    def _zb(i):
        zbuf[pl.ds(i * 16, 16)] = jnp.zeros((16,), jnp.float32)

    pltpu.sync_copy(zbuf, shared.at[pl.ds(s * NSLC, NSLC)])


def _stage_start(ei_hbm, ew_hbm, rcv, ewv, t, sem):
    base = (78 * t + jnp.maximum(t - 28, 0)) * 128
    return [
        pltpu.async_copy(ei_hbm.at[pl.ds(0, 2), pl.ds(base, ECP)], rcv, sem),
        pltpu.async_copy(ew_hbm.at[pl.ds(base, ECP)], ewv, sem),
    ]


def _stage_finish(cps, ewv, t):
    for cp in cps:
        cp.wait()

    @pl.when(t < 28)
    def _():
        for k in range(8):
            ewv[pl.ds((RC - 1) * 128 + k * 16, 16)] = jnp.zeros(
                (16,), jnp.float32)


def _writeout(shared, out0, out1, c, s):
    @pl.when(c == 0)
    def _():
        pltpu.sync_copy(shared.at[pl.ds(s * NSLC, NSLC)],
                        out0.at[pl.ds(s * NSLC, NSLC)])

    @pl.when(c == 1)
    def _():
        pltpu.sync_copy(shared.at[pl.ds(s * NSLC, NSLC)],
                        out1.at[pl.ds(s * NSLC, NSLC)])


@functools.partial(
    pl.kernel,
    out_type=[jax.ShapeDtypeStruct((NPAD,), jnp.float32)] * 2,
    mesh=_mesh,
    scratch_types=[
        pltpu.VMEM((2, ECP), jnp.int32),
        pltpu.VMEM((RC, 128), jnp.int32),
        pltpu.VMEM((ECP,), jnp.float32),
        pltpu.VMEM_SHARED((NPAD,), jnp.float32),
        pltpu.VMEM((NSLC,), jnp.float32),
        pltpu.SemaphoreType.DMA,
    ],
    compiler_params=pltpu.CompilerParams(needs_layout_passes=False),
)
def _sc_deg(ei_hbm, ew_hbm, degp0, degp1, rcv, col2d, ewv, shared, zbuf, sem):
    c = lax.axis_index("c")
    s = lax.axis_index("s")
    t = c * 16 + s
    scps = _stage_start(ei_hbm, ew_hbm, rcv, ewv, t, sem)
    _zero_shared(zbuf, shared, s)
    _stage_finish(scps, ewv, t)
    plsc.subcore_barrier()
    cps = []
    for lo, hi in [(0, 20), (20, 40), (40, 60), (60, RC)]:
        @plsc.parallel_loop(lo * 8, hi * 8, unroll=8)
        def _rw(i):
            col2d[i >> 3, pl.ds((i & 7) * 16, 16)] = rcv[1, pl.ds(i * 16, 16)]

        for j in range(lo, hi):
            cps.append(pltpu.async_copy(
                ewv.at[pl.ds(j * 128, 128)], shared.at[col2d.at[j]], sem,
                add=True))
    for cp in cps:
        cp.wait()
    plsc.subcore_barrier()
    _writeout(shared, degp0, degp1, c, s)


@functools.partial(
    pl.kernel,
    out_type=[jax.ShapeDtypeStruct((NPAD,), jnp.float32)] * 6,
    mesh=_mesh,
    scratch_types=[
        pltpu.VMEM((2, ECP), jnp.int32),
        pltpu.VMEM((RC, 128), jnp.int32),
        pltpu.VMEM((ECP,), jnp.float32),
        pltpu.VMEM((NPAD,), jnp.float32),
        pltpu.VMEM_SHARED((NPAD,), jnp.float32),
        pltpu.VMEM((NSLC,), jnp.float32),
        pltpu.VMEM((NSLC,), jnp.float32),
        pltpu.VMEM((NSLC,), jnp.float32),
        pltpu.VMEM((NSLC,), jnp.float32),
        pltpu.SemaphoreType.DMA,
    ],
    compiler_params=pltpu.CompilerParams(needs_layout_passes=False),
)
def _sc_msg(ei_hbm, ew_hbm, h_hbm, d0_hbm, d1_hbm,
            sp0, sp1, g0, g1, dinv0, self0,
            rcv, col2d, ewv, gv, shared, zbuf, hsl, d0s, d1s, sem):
    c = lax.axis_index("c")
    s = lax.axis_index("s")
    t = c * 16 + s
    nsl = pl.ds(s * NSLC, NSLC)
    scps = _stage_start(ei_hbm, ew_hbm, rcv, ewv, t, sem)
    scps.append(pltpu.async_copy(h_hbm.at[nsl], hsl, sem))
    scps.append(pltpu.async_copy(d0_hbm.at[nsl], d0s, sem))
    scps.append(pltpu.async_copy(d1_hbm.at[nsl], d1s, sem))
    _zero_shared(zbuf, shared, s)
    _stage_finish(scps, ewv, t)

    @plsc.parallel_loop(0, NSLC // 16, unroll=4)
    def _prep(i):
        sl = pl.ds(i * 16, 16)
        deg = 1.0 + d0s[sl] + d1s[sl]
        seed = plsc.bitcast(
            0x5F3759DF - (plsc.bitcast(deg, jnp.int32) >> 1), jnp.float32)
        y = seed * (1.5 - 0.5 * deg * seed * seed)
        y = y * (1.5 - 0.5 * deg * y * y)
        y = y * (1.5 - 0.5 * deg * y * y)
        gs = hsl[sl] * y
        hsl[sl] = gs
        d0s[sl] = y
        d1s[sl] = gs * y

    @pl.when(c == 0)
    def _():
        pltpu.sync_copy(hsl, g0.at[nsl])
        pltpu.sync_copy(d0s, dinv0.at[nsl])
        pltpu.sync_copy(d1s, self0.at[nsl])

    @pl.when(c == 1)
    def _():
        pltpu.sync_copy(hsl, g1.at[nsl])

    plsc.subcore_barrier()

    @pl.when(c == 0)
    def _():
        pltpu.sync_copy(g0, gv)

    @pl.when(c == 1)
    def _():
        pltpu.sync_copy(g1, gv)

    batches = [(0, 20), (20, 40), (40, 60), (60, RC)]
    cps = []
    for lo, hi in batches:
        @plsc.parallel_loop(lo * 8, hi * 8, unroll=8)
        def _gm(i):
            sl = pl.ds(i * 16, 16)
            ii = rcv[0, sl]
            vals = plsc.load_gather(gv, [ii])
            ewv[sl] = vals * ewv[sl]
            col2d[i >> 3, pl.ds((i & 7) * 16, 16)] = rcv[1, sl]

        for j in range(lo, hi):
            cps.append(pltpu.async_copy(
                ewv.at[pl.ds(j * 128, 128)], shared.at[col2d.at[j]], sem,
                add=True))
    for cp in cps:
        cp.wait()
    plsc.subcore_barrier()
    _writeout(shared, sp0, sp1, c, s)


def _tc_mm_body(x_ref, w_ref, h_ref):
    h_ref[...] = lax.dot_general(x_ref[...], w_ref[...],
                                 (((2,), (0,)), ((), ())),
                                 preferred_element_type=jnp.float32)


def _tc_final_body(s0_ref, s1_ref, dinv_ref, self_ref, b_ref, out_ref):
    v = ((s0_ref[...] + s1_ref[...]) * dinv_ref[...] + self_ref[...]
         + b_ref[...])
    sp = jnp.maximum(v, 0.0) + jnp.log1p(jnp.exp(-jnp.abs(v)))
    out_ref[...] = v * jnp.tanh(sp)


def kernel(x, edge_index, edge_attr, W, b):
    ei = edge_index.astype(jnp.int32)
    ea = edge_attr.astype(jnp.float32)

    degp0, degp1 = _sc_deg(ei, ea)

    x3 = jnp.concatenate(
        [x, jnp.zeros((NPAD - N_NODES, D_FEAT), jnp.float32)]
    ).reshape(NROWS, 128, 128)
    w1 = W.reshape(128)
    b2 = jnp.broadcast_to(b.reshape(1, 1), (1, 128))

    h3 = pl.pallas_call(
        _tc_mm_body,
        out_shape=jax.ShapeDtypeStruct((NROWS, 128), jnp.float32),
    )(x3, w1)

    sp0, sp1, _g0, _g1, dinv0, self0 = _sc_msg(
        ei, ea, h3.reshape(NPAD), degp0, degp1)

    out = pl.pallas_call(
        _tc_final_body,
        out_shape=jax.ShapeDtypeStruct((NROWS, 128), jnp.float32),
    )(sp0.reshape(NROWS, 128), sp1.reshape(NROWS, 128),
      dinv0.reshape(NROWS, 128), self0.reshape(NROWS, 128), b2)

    return out.reshape(1, NPAD)[:, :N_NODES]


# speedup vs baseline: 1.0526x; 1.0526x over previous
import functools

import jax
import jax.numpy as jnp
from jax import lax
from jax.experimental import pallas as pl
from jax.experimental.pallas import tpu as pltpu
from jax.experimental.pallas import tpu_sc as plsc

N_NODES = 10000
D_FEAT = 128
N_EDGES = 320000

NPAD = 10240
NROWS = NPAD // 128
RC = 79
ECP = RC * 128
NCHUNKS = N_EDGES // 128
NSLC = NPAD // 16

_mesh = plsc.VectorSubcoreMesh(core_axis_name="c", subcore_axis_name="s")


def _zero_shared(zbuf, shared, s):
    @plsc.parallel_loop(0, NSLC // 16, unroll=4)
    def _zb(i):
        zbuf[pl.ds(i * 16, 16)] = jnp.zeros((16,), jnp.float32)

    pltpu.sync_copy(zbuf, shared.at[pl.ds(s * NSLC, NSLC)])


def _stage_start(ei_hbm, ew_hbm, rcv, ewv, t, sem):
    base = (78 * t + jnp.maximum(t - 28, 0)) * 128
    return [
        pltpu.async_copy(ei_hbm.at[pl.ds(0, 2), pl.ds(base, ECP)], rcv, sem),
        pltpu.async_copy(ew_hbm.at[pl.ds(base, ECP)], ewv, sem),
    ]


def _stage_finish(cps, ewv, t):
    for cp in cps:
        cp.wait()

    @pl.when(t < 28)
    def _():
        for k in range(8):
            ewv[pl.ds((RC - 1) * 128 + k * 16, 16)] = jnp.zeros(
                (16,), jnp.float32)


def _writeout(shared, out0, out1, c, s):
    @pl.when(c == 0)
    def _():
        pltpu.sync_copy(shared.at[pl.ds(s * NSLC, NSLC)],
                        out0.at[pl.ds(s * NSLC, NSLC)])

    @pl.when(c == 1)
    def _():
        pltpu.sync_copy(shared.at[pl.ds(s * NSLC, NSLC)],
                        out1.at[pl.ds(s * NSLC, NSLC)])


@functools.partial(
    pl.kernel,
    out_type=[jax.ShapeDtypeStruct((NPAD,), jnp.float32)] * 2,
    mesh=_mesh,
    scratch_types=[
        pltpu.VMEM((2, ECP), jnp.int32),
        pltpu.VMEM((RC, 128), jnp.int32),
        pltpu.VMEM((ECP,), jnp.float32),
        pltpu.VMEM_SHARED((NPAD,), jnp.float32),
        pltpu.VMEM((NSLC,), jnp.float32),
        pltpu.SemaphoreType.DMA,
    ],
    compiler_params=pltpu.CompilerParams(needs_layout_passes=False),
)
def _sc_deg(ei_hbm, ew_hbm, degp0, degp1, rcv, col2d, ewv, shared, zbuf, sem):
    c = lax.axis_index("c")
    s = lax.axis_index("s")
    t = c * 16 + s
    scps = _stage_start(ei_hbm, ew_hbm, rcv, ewv, t, sem)
    _zero_shared(zbuf, shared, s)
    _stage_finish(scps, ewv, t)
    plsc.subcore_barrier()
    cps = []
    for lo, hi in [(0, 20), (20, 40), (40, 60), (60, RC)]:
        @plsc.parallel_loop(lo * 8, hi * 8, unroll=8)
        def _rw(i):
            col2d[i >> 3, pl.ds((i & 7) * 16, 16)] = rcv[1, pl.ds(i * 16, 16)]

        for j in range(lo, hi):
            cps.append(pltpu.async_copy(
                ewv.at[pl.ds(j * 128, 128)], shared.at[col2d.at[j]], sem,
                add=True))
    for cp in cps:
        cp.wait()
    plsc.subcore_barrier()
    _writeout(shared, degp0, degp1, c, s)


@functools.partial(
    pl.kernel,
    out_type=[jax.ShapeDtypeStruct((NPAD,), jnp.float32)] * 2,
    mesh=_mesh,
    scratch_types=[
        pltpu.VMEM((2, ECP), jnp.int32),
        pltpu.VMEM((RC, 128), jnp.int32),
        pltpu.VMEM((ECP,), jnp.float32),
        pltpu.VMEM((NPAD,), jnp.float32),
        pltpu.VMEM_SHARED((NPAD,), jnp.float32),
        pltpu.VMEM((NSLC,), jnp.float32),
        pltpu.SemaphoreType.DMA,
    ],
    compiler_params=pltpu.CompilerParams(needs_layout_passes=False),
)
def _sc_msg(ei_hbm, ew_hbm, g_hbm, sp0, sp1,
            rcv, col2d, ewv, gv, shared, zbuf, sem):
    c = lax.axis_index("c")
    s = lax.axis_index("s")
    t = c * 16 + s
    scps = _stage_start(ei_hbm, ew_hbm, rcv, ewv, t, sem)
    scps.append(pltpu.async_copy(g_hbm, gv, sem))
    _zero_shared(zbuf, shared, s)
    _stage_finish(scps, ewv, t)
    plsc.subcore_barrier()
    batches = [(0, 20), (20, 40), (40, 60), (60, RC)]
    cps = []
    for lo, hi in batches:
        @plsc.parallel_loop(lo * 8, hi * 8, unroll=8)
        def _gm(i):
            sl = pl.ds(i * 16, 16)
            ii = rcv[0, sl]
            vals = plsc.load_gather(gv, [ii])
            ewv[sl] = vals * ewv[sl]
            col2d[i >> 3, pl.ds((i & 7) * 16, 16)] = rcv[1, sl]

        for j in range(lo, hi):
            cps.append(pltpu.async_copy(
                ewv.at[pl.ds(j * 128, 128)], shared.at[col2d.at[j]], sem,
                add=True))
    for cp in cps:
        cp.wait()
    plsc.subcore_barrier()
    _writeout(shared, sp0, sp1, c, s)


def _tc_mm_body(x_ref, w_ref, h_ref):
    h_ref[...] = lax.dot_general(x_ref[...], w_ref[...],
                                 (((2,), (0,)), ((), ())),
                                 preferred_element_type=jnp.float32)


def _tc_prep_body(h_ref, d0_ref, d1_ref, b_ref, g_ref, dinv_ref, self_ref):
    h = h_ref[...]
    deg = 1.0 + d0_ref[...] + d1_ref[...]
    dinv = lax.rsqrt(deg)
    g_ref[...] = h * dinv
    dinv_ref[...] = dinv
    self_ref[...] = h / deg + b_ref[...]


def _tc_final_body(s0_ref, s1_ref, dinv_ref, self_ref, out_ref):
    v = (s0_ref[...] + s1_ref[...]) * dinv_ref[...] + self_ref[...]
    sp = jnp.maximum(v, 0.0) + jnp.log1p(jnp.exp(-jnp.abs(v)))
    out_ref[...] = v * jnp.tanh(sp)


def kernel(x, edge_index, edge_attr, W, b):
    ei = edge_index.astype(jnp.int32)
    ea = edge_attr.astype(jnp.float32)

    degp0, degp1 = _sc_deg(ei, ea)

    x3 = jnp.concatenate(
        [x, jnp.zeros((NPAD - N_NODES, D_FEAT), jnp.float32)]
    ).reshape(NROWS, 128, 128)
    w1 = W.reshape(128)
    b2 = jnp.broadcast_to(b.reshape(1, 1), (1, 128))

    h3 = pl.pallas_call(
        _tc_mm_body,
        out_shape=jax.ShapeDtypeStruct((NROWS, 128), jnp.float32),
    )(x3, w1)

    g, dinv, selfterm = pl.pallas_call(
        _tc_prep_body,
        out_shape=[jax.ShapeDtypeStruct((NROWS, 128), jnp.float32)] * 3,
    )(h3, degp0.reshape(NROWS, 128), degp1.reshape(NROWS, 128), b2)

    sp0, sp1 = _sc_msg(ei, ea, g.reshape(NPAD))

    out = pl.pallas_call(
        _tc_final_body,
        out_shape=jax.ShapeDtypeStruct((NROWS, 128), jnp.float32),
    )(sp0.reshape(NROWS, 128), sp1.reshape(NROWS, 128), dinv, selfterm)

    return out.reshape(1, NPAD)[:, :N_NODES]
